# baseline (speedup 1.0000x reference)
import functools

import jax
import jax.numpy as jnp
from jax import lax
from jax.experimental import pallas as pl
from jax.experimental.pallas import tpu as pltpu
from jax.experimental.pallas import tpu_sc as plsc

N = 10000
D = 128
NV = 8
DV = 16
DOUT = 128
L = 16
NC = 2
NS = 16
NW = NC * NS
NP = 10240
C = 128
NCH0 = 112
NCH1 = 52
NCHT = NCH0 + NCH1
EPAD = NS * NCHT * C
EXTRA = NCH0 * C
RPS = NP // NS

_mesh = plsc.VectorSubcoreMesh(core_axis_name="c", subcore_axis_name="s")
_sc_params = pltpu.CompilerParams(
    needs_layout_passes=False, use_tc_tiling_on_sc=False)


def _proj_body(x_ref, wt_ref, ws_ref, wh_ref, bt_ref, bs_ref, bh_ref,
               ht_ref, hs_ref, hn_ref):
    xv = x_ref[...]
    ht_ref[...] = jnp.maximum(
        jnp.dot(xv, wt_ref[...], preferred_element_type=jnp.float32)
        + bt_ref[...], 0.0)
    hs_ref[...] = jnp.maximum(
        jnp.dot(xv, ws_ref[...], preferred_element_type=jnp.float32)
        + bs_ref[...], 0.0)
    hn_ref[...] = jnp.maximum(
        jnp.dot(xv, wh_ref[...], preferred_element_type=jnp.float32)
        + bh_ref[...], 0.0)


def _project(xp, Wt, Ws, Wh, bt, bs, bh):
    BR = 512
    full = lambda i: (0, 0)
    row = lambda i: (i, 0)
    return pl.pallas_call(
        _proj_body,
        grid=(NP // BR,),
        in_specs=[
            pl.BlockSpec((BR, D), row),
            pl.BlockSpec((D, NV * DV), full),
            pl.BlockSpec((D, NV * DV), full),
            pl.BlockSpec((D, NV * DV), full),
            pl.BlockSpec((1, NV * DV), full),
            pl.BlockSpec((1, NV * DV), full),
            pl.BlockSpec((1, NV * DV), full),
        ],
        out_specs=[pl.BlockSpec((BR, NV * DV), row)] * 3,
        out_shape=[jax.ShapeDtypeStruct((NP, NV * DV), jnp.float32)] * 3,
    )(xp, Wt, Ws, Wh, bt, bs, bh)


@functools.partial(
    pl.kernel,
    out_type=[
        jax.ShapeDtypeStruct((NS * NCHT, L, C), jnp.float32),
        jax.ShapeDtypeStruct((NP, L), jnp.float32),
        jax.ShapeDtypeStruct((NP, L), jnp.float32),
    ],
    mesh=_mesh,
    scratch_types=[
        pltpu.VMEM((NCH0, C), jnp.int32),
        pltpu.VMEM((NCH0, C), jnp.int32),
        pltpu.VMEM((C, D), jnp.float32),
        pltpu.VMEM((C, D), jnp.float32),
        pltpu.VMEM((C, D), jnp.float32),
        pltpu.VMEM((C, D), jnp.float32),
        pltpu.VMEM((L, C), jnp.float32),
        pltpu.VMEM((L, C), jnp.float32),
        pltpu.VMEM((C, L), jnp.float32),
        pltpu.VMEM((C, L), jnp.float32),
        pltpu.VMEM_SHARED((NP, L), jnp.float32),
        pltpu.SemaphoreType.DMA,
        pltpu.SemaphoreType.DMA,
        pltpu.SemaphoreType.DMA,
        pltpu.SemaphoreType.DMA,
        pltpu.SemaphoreType.DMA,
        pltpu.SemaphoreType.DMA,
        pltpu.SemaphoreType.DMA,
        pltpu.SemaphoreType.DMA,
    ],
    compiler_params=_sc_params,
)
def _pass_a(ht, hs, src2, dst2, ec_out, norm0, norm1,
            ixd, ixs, ab0, ab1, bb0, bb1, et0, et1, rb0, rb1, norm_sh,
            sa0, sa1, sb0, sb1, se0, se1, sc0, sc1):
    cid = lax.axis_index("c")
    sid = lax.axis_index("s")
    nch = jnp.where(cid == 0, NCH0, NCH1)
    cbase = jnp.where(cid == 0, sid * NCH0, NS * NCH0 + sid * NCH1)
    iota = lax.iota(jnp.int32, L)
    zero16 = jnp.zeros((L,), jnp.float32)
    rot = [(iota + d) % L for d in range(L)]

    pltpu.sync_copy(dst2.at[pl.ds(cbase, NCH0)], ixd)
    pltpu.sync_copy(src2.at[pl.ds(cbase, NCH0)], ixs)

    def zrow(i, carry):
        rb0[i] = zero16
        return carry
    lax.fori_loop(0, C, zrow, 0)
    for i in range(RPS // C):
        pltpu.sync_copy(rb0, norm_sh.at[pl.ds(sid * RPS + i * C, C)])
    for et in (et0, et1):
        def zpad(g, carry):
            for k in range(NV, L):
                et[k, pl.ds(g * L, L)] = zero16
            return carry
        lax.fori_loop(0, C // L, zpad, 0)
    plsc.subcore_barrier()

    def issue(j, ab, bb, sa, sb):
        pltpu.async_copy(ht.at[ixd.at[j]], ab, sa)
        pltpu.async_copy(hs.at[ixs.at[j]], bb, sb)

    def wait(j, ab, bb, sa, sb):
        pltpu.make_async_copy(ht.at[ixd.at[j]], ab, sa).wait()
        pltpu.make_async_copy(hs.at[ixs.at[j]], bb, sb).wait()

    def compute(ab, bb, et, rb):
        def group(g, gc):
            rows = g * L + iota
            cols = g * L + iota
            accs = [zero16] * NV
            for d in range(DV):
                for k in range(NV):
                    cv = k * DV + rot[d]
                    accs[k] = accs[k] + (plsc.load_gather(ab, [rows, cv])
                                         * plsc.load_gather(bb, [rows, cv]))
            for k in range(NV):
                et[k, pl.ds(g * L, L)] = jnp.exp(accs[k])
            for d in range(L):
                v = plsc.load_gather(et, [rot[d], cols])
                plsc.store_scatter(rb, [cols, rot[d]], v)
            return gc
        lax.fori_loop(0, C // L, group, 0)

    def writeout(j, et, rb, se, sc):
        pltpu.async_copy(et, ec_out.at[cbase + j], se)
        pltpu.async_copy(rb, norm_sh.at[ixd.at[j]], sc, add=True)

    def drain(j, et, rb, se, sc):
        pltpu.make_async_copy(et, ec_out.at[cbase + j], se).wait()
        pltpu.make_async_copy(rb, norm_sh.at[ixd.at[j]], sc).wait()

    issue(0, ab0, bb0, sa0, sb0)
    issue(1, ab1, bb1, sa1, sb1)

    def pair(jj, carry):
        c0 = 2 * jj
        c1 = c0 + 1
        wait(c0, ab0, bb0, sa0, sb0)

        @pl.when(jj > 0)
        def _():
            drain(c0 - 2, et0, rb0, se0, sc0)
        compute(ab0, bb0, et0, rb0)
        writeout(c0, et0, rb0, se0, sc0)

        @pl.when(c0 + 2 < nch)
        def _():
            issue(c0 + 2, ab0, bb0, sa0, sb0)

        wait(c1, ab1, bb1, sa1, sb1)

        @pl.when(jj > 0)
        def _():
            drain(c1 - 2, et1, rb1, se1, sc1)
        compute(ab1, bb1, et1, rb1)
        writeout(c1, et1, rb1, se1, sc1)

        @pl.when(c1 + 2 < nch)
        def _():
            issue(c1 + 2, ab1, bb1, sa1, sb1)
        return carry
    lax.fori_loop(0, nch // 2, pair, 0)
    drain(nch - 2, et0, rb0, se0, sc0)
    drain(nch - 1, et1, rb1, se1, sc1)

    plsc.subcore_barrier()
    rs = sid * RPS

    @pl.when(cid == 0)
    def _():
        pltpu.sync_copy(norm_sh.at[pl.ds(rs, RPS)], norm0.at[pl.ds(rs, RPS)])

    @pl.when(cid == 1)
    def _():
        pltpu.sync_copy(norm_sh.at[pl.ds(rs, RPS)], norm1.at[pl.ds(rs, RPS)])


@functools.partial(
    pl.kernel,
    out_type=[
        jax.ShapeDtypeStruct((NP, L), jnp.float32),
        jax.ShapeDtypeStruct((NP, L), jnp.float32),
    ],
    mesh=_mesh,
    scratch_types=[
        pltpu.VMEM((NCH0, C), jnp.int32),
        pltpu.VMEM((NCH0, C), jnp.int32),
        pltpu.VMEM((C, D), jnp.float32),
        pltpu.VMEM((C, D), jnp.float32),
        pltpu.VMEM((C, L), jnp.float32),
        pltpu.VMEM((C, L), jnp.float32),
        pltpu.VMEM((L, C), jnp.float32),
        pltpu.VMEM((L, C), jnp.float32),
        pltpu.VMEM((L, C), jnp.float32),
        pltpu.VMEM((L, C), jnp.float32),
        pltpu.VMEM((C, L), jnp.float32),
        pltpu.VMEM((C, L), jnp.float32),
        pltpu.VMEM_SHARED((NP, L), jnp.float32),
        pltpu.VMEM_SHARED((NP, L), jnp.float32),
        pltpu.SemaphoreType.DMA,
        pltpu.SemaphoreType.DMA,
        pltpu.SemaphoreType.DMA,
        pltpu.SemaphoreType.DMA,
        pltpu.SemaphoreType.DMA,
        pltpu.SemaphoreType.DMA,
        pltpu.SemaphoreType.DMA,
        pltpu.SemaphoreType.DMA,
    ],
    compiler_params=_sc_params,
)
def _pass_b(hn, src2, dst2, ec_in, norm0, norm1, agg0, agg1,
            ixd, ixs, bb0, bb1, nm0, nm1, nt0, nt1, ec0, ec1, mb0, mb1,
            rcp_sh, agg_sh, sa0, sa1, sn0, sn1, se0, se1, sc0, sc1):
    cid = lax.axis_index("c")
    sid = lax.axis_index("s")
    nch = jnp.where(cid == 0, NCH0, NCH1)
    cbase = jnp.where(cid == 0, sid * NCH0, NS * NCH0 + sid * NCH1)
    iota = lax.iota(jnp.int32, L)
    zero16 = jnp.zeros((L,), jnp.float32)
    rot = [(iota + d) % L for d in range(L)]
    rs = sid * RPS

    pltpu.sync_copy(dst2.at[pl.ds(cbase, NCH0)], ixd)
    pltpu.sync_copy(src2.at[pl.ds(cbase, NCH0)], ixs)

    for i in range(RPS // C):
        pltpu.sync_copy(norm0.at[pl.ds(rs + i * C, C)], mb0)
        pltpu.sync_copy(norm1.at[pl.ds(rs + i * C, C)], nm0)

        def rcprow(r, carry):
            mb0[r] = 1.0 / (mb0[r] + nm0[r] + 1e-12)
            return carry
        lax.fori_loop(0, C, rcprow, 0)
        pltpu.sync_copy(mb0, rcp_sh.at[pl.ds(rs + i * C, C)])

    def zrow(i, carry):
        mb0[i] = zero16
        return carry
    lax.fori_loop(0, C, zrow, 0)
    for i in range(RPS // C):
        pltpu.sync_copy(mb0, agg_sh.at[pl.ds(rs + i * C, C)])
    plsc.subcore_barrier()

    def issue(j, bb, nm, ec, sa, sn, se):
        pltpu.async_copy(hn.at[ixs.at[j]], bb, sa)
        pltpu.async_copy(rcp_sh.at[ixd.at[j]], nm, sn)
        pltpu.async_copy(ec_in.at[cbase + j], ec, se)

    def wait(j, bb, nm, ec, sa, sn, se):
        pltpu.make_async_copy(hn.at[ixs.at[j]], bb, sa).wait()
        pltpu.make_async_copy(rcp_sh.at[ixd.at[j]], nm, sn).wait()
        pltpu.make_async_copy(ec_in.at[cbase + j], ec, se).wait()

    def compute(bb, nm, nt, ec, mb):
        def group(g, gc):
            rows = g * L + iota
            for d in range(L):
                v = plsc.load_gather(nm, [rows, rot[d]])
                plsc.store_scatter(nt, [rot[d], rows], v)
            als = []
            for k in range(NV):
                s = pl.ds(g * L, L)
                als.append(ec[k, s] * nt[k, s] * (1.0 / NV))
            accs = [zero16] * DV
            for k in range(NV):
                for d in range(DV):
                    cv = k * DV + rot[d]
                    accs[d] = accs[d] + als[k] * plsc.load_gather(bb, [rows, cv])
            for d in range(DV):
                plsc.store_scatter(mb, [rows, rot[d]], accs[d])
            return gc
        lax.fori_loop(0, C // L, group, 0)

    issue(0, bb0, nm0, ec0, sa0, sn0, se0)
    issue(1, bb1, nm1, ec1, sa1, sn1, se1)

    def pair(jj, carry):
        c0 = 2 * jj
        c1 = c0 + 1
        wait(c0, bb0, nm0, ec0, sa0, sn0, se0)

        @pl.when(jj > 0)
        def _():
            pltpu.make_async_copy(mb0, agg_sh.at[ixd.at[c0 - 2]], sc0).wait()
        compute(bb0, nm0, nt0, ec0, mb0)
        pltpu.async_copy(mb0, agg_sh.at[ixd.at[c0]], sc0, add=True)

        @pl.when(c0 + 2 < nch)
        def _():
            issue(c0 + 2, bb0, nm0, ec0, sa0, sn0, se0)

        wait(c1, bb1, nm1, ec1, sa1, sn1, se1)

        @pl.when(jj > 0)
        def _():
            pltpu.make_async_copy(mb1, agg_sh.at[ixd.at[c1 - 2]], sc1).wait()
        compute(bb1, nm1, nt1, ec1, mb1)
        pltpu.async_copy(mb1, agg_sh.at[ixd.at[c1]], sc1, add=True)

        @pl.when(c1 + 2 < nch)
        def _():
            issue(c1 + 2, bb1, nm1, ec1, sa1, sn1, se1)
        return carry
    lax.fori_loop(0, nch // 2, pair, 0)
    pltpu.make_async_copy(mb0, agg_sh.at[ixd.at[nch - 2]], sc0).wait()
    pltpu.make_async_copy(mb1, agg_sh.at[ixd.at[nch - 1]], sc1).wait()

    plsc.subcore_barrier()

    @pl.when(cid == 0)
    def _():
        pltpu.sync_copy(agg_sh.at[pl.ds(rs, RPS)], agg0.at[pl.ds(rs, RPS)])

    @pl.when(cid == 1)
    def _():
        pltpu.sync_copy(agg_sh.at[pl.ds(rs, RPS)], agg1.at[pl.ds(rs, RPS)])


def _out_body(a0_ref, a1_ref, w_ref, b_ref, o_ref):
    agg = a0_ref[...] + a1_ref[...]
    o_ref[...] = jnp.maximum(
        jnp.dot(agg, w_ref[...], preferred_element_type=jnp.float32)
        + b_ref[...], 0.0)


def _final(a0, a1, W_out, b_out):
    BR = 512
    full = lambda i: (0, 0)
    row = lambda i: (i, 0)
    return pl.pallas_call(
        _out_body,
        grid=(NP // BR,),
        in_specs=[
            pl.BlockSpec((BR, L), row),
            pl.BlockSpec((BR, L), row),
            pl.BlockSpec((DV, DOUT), full),
            pl.BlockSpec((1, DOUT), full),
        ],
        out_specs=pl.BlockSpec((BR, DOUT), row),
        out_shape=jax.ShapeDtypeStruct((NP, DOUT), jnp.float32),
    )(a0, a1, W_out, b_out.reshape(1, -1))


def kernel(x, edge_index, W_target, b_target, W_source, b_source,
           W_hidden, b_hidden, W_out, b_out):
    xp = jnp.pad(x, ((0, NP - N), (0, 0)))
    ht, hs, hn = _project(
        xp, W_target, W_source, W_hidden,
        b_target.reshape(1, -1), b_source.reshape(1, -1),
        b_hidden.reshape(1, -1))

    ne = edge_index.shape[1]
    loops = jnp.arange(N, dtype=jnp.int32)
    padi = jnp.full((EPAD - ne - N + EXTRA,), NP - 1, jnp.int32)
    src2 = jnp.concatenate(
        [edge_index[0].astype(jnp.int32), loops, padi]).reshape(-1, C)
    dst2 = jnp.concatenate(
        [edge_index[1].astype(jnp.int32), loops, padi]).reshape(-1, C)

    ec, n0, n1 = _pass_a(ht, hs, src2, dst2)
    a0, a1 = _pass_b(hn, src2, dst2, ec, n0, n1)
    out = _final(a0, a1, W_out, b_out)
    return out[:N]


# speedup vs baseline: 1.2319x; 1.2319x over previous
import functools

import jax
import jax.numpy as jnp
from jax import lax
from jax.experimental import pallas as pl
from jax.experimental.pallas import tpu as pltpu
from jax.experimental.pallas import tpu_sc as plsc

N = 10000
D = 128
NV = 8
DV = 16
DOUT = 128
L = 16
NC = 2
NS = 16
NW = NC * NS
NP = 10240
C = 128
NCH0A = 100
NCH0B = 108
NCHT = 164
NCH1A = NCHT - NCH0A
NCH1B = NCHT - NCH0B
EPAD = NS * NCHT * C
EXTRA = max(NCH0A, NCH0B) * C
RPS = NP // NS

_mesh = plsc.VectorSubcoreMesh(core_axis_name="c", subcore_axis_name="s")
_sc_params = pltpu.CompilerParams(
    needs_layout_passes=False, use_tc_tiling_on_sc=False)


def _proj_body(x_ref, wt_ref, ws_ref, wh_ref, bt_ref, bs_ref, bh_ref,
               ht_ref, hs_ref, hn_ref):
    xv = x_ref[...]
    ht_ref[...] = jnp.maximum(
        jnp.dot(xv, wt_ref[...], preferred_element_type=jnp.float32)
        + bt_ref[...], 0.0)
    hs_ref[...] = jnp.maximum(
        jnp.dot(xv, ws_ref[...], preferred_element_type=jnp.float32)
        + bs_ref[...], 0.0)
    hn_ref[...] = jnp.maximum(
        jnp.dot(xv, wh_ref[...], preferred_element_type=jnp.float32)
        + bh_ref[...], 0.0)


def _project(xp, Wt, Ws, Wh, bt, bs, bh):
    BR = 512
    full = lambda i: (0, 0)
    row = lambda i: (i, 0)
    return pl.pallas_call(
        _proj_body,
        grid=(NP // BR,),
        in_specs=[
            pl.BlockSpec((BR, D), row),
            pl.BlockSpec((D, NV * DV), full),
            pl.BlockSpec((D, NV * DV), full),
            pl.BlockSpec((D, NV * DV), full),
            pl.BlockSpec((1, NV * DV), full),
            pl.BlockSpec((1, NV * DV), full),
            pl.BlockSpec((1, NV * DV), full),
        ],
        out_specs=[pl.BlockSpec((BR, NV * DV), row)] * 3,
        out_shape=[jax.ShapeDtypeStruct((NP, NV * DV), jnp.float32)] * 3,
    )(xp, Wt, Ws, Wh, bt, bs, bh)


@functools.partial(
    pl.kernel,
    out_type=[
        jax.ShapeDtypeStruct((NS * NCHT, L, C), jnp.float32),
        jax.ShapeDtypeStruct((NP, L), jnp.float32),
        jax.ShapeDtypeStruct((NP, L), jnp.float32),
    ],
    mesh=_mesh,
    scratch_types=[
        pltpu.VMEM((NCH0A, C), jnp.int32),
        pltpu.VMEM((NCH0A, C), jnp.int32),
        pltpu.VMEM((C, D), jnp.float32),
        pltpu.VMEM((C, D), jnp.float32),
        pltpu.VMEM((C, D), jnp.float32),
        pltpu.VMEM((C, D), jnp.float32),
        pltpu.VMEM((L, C), jnp.float32),
        pltpu.VMEM((L, C), jnp.float32),
        pltpu.VMEM((C, L), jnp.float32),
        pltpu.VMEM((C, L), jnp.float32),
        pltpu.VMEM_SHARED((NP, L), jnp.float32),
        pltpu.SemaphoreType.DMA,
        pltpu.SemaphoreType.DMA,
        pltpu.SemaphoreType.DMA,
        pltpu.SemaphoreType.DMA,
        pltpu.SemaphoreType.DMA,
        pltpu.SemaphoreType.DMA,
        pltpu.SemaphoreType.DMA,
        pltpu.SemaphoreType.DMA,
    ],
    compiler_params=_sc_params,
)
def _pass_a(ht, hs, src2, dst2, ec_out, norm0, norm1,
            ixd, ixs, ab0, ab1, bb0, bb1, et0, et1, rb0, rb1, norm_sh,
            sa0, sa1, sb0, sb1, se0, se1, sc0, sc1):
    cid = lax.axis_index("c")
    sid = lax.axis_index("s")
    nch = jnp.where(cid == 0, NCH0A, NCH1A)
    cbase = jnp.where(cid == 0, sid * NCH0A, NS * NCH0A + sid * NCH1A)
    iota = lax.iota(jnp.int32, L)
    zero16 = jnp.zeros((L,), jnp.float32)
    rot = [(iota + d) % L for d in range(L)]

    pltpu.sync_copy(dst2.at[pl.ds(cbase, NCH0A)], ixd)
    pltpu.sync_copy(src2.at[pl.ds(cbase, NCH0A)], ixs)

    def zrow(i, carry):
        rb0[i] = zero16
        return carry
    lax.fori_loop(0, C, zrow, 0)
    for i in range(RPS // C):
        pltpu.sync_copy(rb0, norm_sh.at[pl.ds(sid * RPS + i * C, C)])
    for et in (et0, et1):
        def zpad(g, carry):
            for k in range(NV, L):
                et[k, pl.ds(g * L, L)] = zero16
            return carry
        lax.fori_loop(0, C // L, zpad, 0)
    plsc.subcore_barrier()

    def issue(j, ab, bb, sa, sb):
        pltpu.async_copy(ht.at[ixd.at[j]], ab, sa)
        pltpu.async_copy(hs.at[ixs.at[j]], bb, sb)

    def wait(j, ab, bb, sa, sb):
        pltpu.make_async_copy(ht.at[ixd.at[j]], ab, sa).wait()
        pltpu.make_async_copy(hs.at[ixs.at[j]], bb, sb).wait()

    def compute(ab, bb, et, rb):
        def group(g, gc):
            rows = g * L + iota
            cols = g * L + iota
            for k in range(NV):
                acc = zero16
                for d in range(DV):
                    cv = k * DV + rot[d]
                    acc = acc + (plsc.load_gather(ab, [rows, cv])
                                 * plsc.load_gather(bb, [rows, cv]))
                et[k, pl.ds(g * L, L)] = jnp.exp(acc)
            for d in range(L):
                v = plsc.load_gather(et, [rot[d], cols])
                plsc.store_scatter(rb, [cols, rot[d]], v)
            return gc
        lax.fori_loop(0, C // L, group, 0)

    def writeout(j, et, rb, se, sc):
        pltpu.async_copy(et, ec_out.at[cbase + j], se)
        pltpu.async_copy(rb, norm_sh.at[ixd.at[j]], sc, add=True)

    def drain(j, et, rb, se, sc):
        pltpu.make_async_copy(et, ec_out.at[cbase + j], se).wait()
        pltpu.make_async_copy(rb, norm_sh.at[ixd.at[j]], sc).wait()

    issue(0, ab0, bb0, sa0, sb0)
    issue(1, ab1, bb1, sa1, sb1)

    def pair(jj, carry):
        c0 = 2 * jj
        c1 = c0 + 1
        wait(c0, ab0, bb0, sa0, sb0)

        @pl.when(jj > 0)
        def _():
            drain(c0 - 2, et0, rb0, se0, sc0)
        compute(ab0, bb0, et0, rb0)
        writeout(c0, et0, rb0, se0, sc0)

        @pl.when(c0 + 2 < nch)
        def _():
            issue(c0 + 2, ab0, bb0, sa0, sb0)

        wait(c1, ab1, bb1, sa1, sb1)

        @pl.when(jj > 0)
        def _():
            drain(c1 - 2, et1, rb1, se1, sc1)
        compute(ab1, bb1, et1, rb1)
        writeout(c1, et1, rb1, se1, sc1)

        @pl.when(c1 + 2 < nch)
        def _():
            issue(c1 + 2, ab1, bb1, sa1, sb1)
        return carry
    lax.fori_loop(0, nch // 2, pair, 0)
    drain(nch - 2, et0, rb0, se0, sc0)
    drain(nch - 1, et1, rb1, se1, sc1)

    plsc.subcore_barrier()
    rs = sid * RPS

    @pl.when(cid == 0)
    def _():
        pltpu.sync_copy(norm_sh.at[pl.ds(rs, RPS)], norm0.at[pl.ds(rs, RPS)])

    @pl.when(cid == 1)
    def _():
        pltpu.sync_copy(norm_sh.at[pl.ds(rs, RPS)], norm1.at[pl.ds(rs, RPS)])


@functools.partial(
    pl.kernel,
    out_type=[
        jax.ShapeDtypeStruct((NP, L), jnp.float32),
        jax.ShapeDtypeStruct((NP, L), jnp.float32),
    ],
    mesh=_mesh,
    scratch_types=[
        pltpu.VMEM((NCH0B, C), jnp.int32),
        pltpu.VMEM((NCH0B, C), jnp.int32),
        pltpu.VMEM((C, D), jnp.float32),
        pltpu.VMEM((C, D), jnp.float32),
        pltpu.VMEM((C, L), jnp.float32),
        pltpu.VMEM((C, L), jnp.float32),
        pltpu.VMEM((L, C), jnp.float32),
        pltpu.VMEM((L, C), jnp.float32),
        pltpu.VMEM((L, C), jnp.float32),
        pltpu.VMEM((L, C), jnp.float32),
        pltpu.VMEM((C, L), jnp.float32),
        pltpu.VMEM((C, L), jnp.float32),
        pltpu.VMEM_SHARED((NP, L), jnp.float32),
        pltpu.VMEM_SHARED((NP, L), jnp.float32),
        pltpu.SemaphoreType.DMA,
        pltpu.SemaphoreType.DMA,
        pltpu.SemaphoreType.DMA,
        pltpu.SemaphoreType.DMA,
        pltpu.SemaphoreType.DMA,
        pltpu.SemaphoreType.DMA,
        pltpu.SemaphoreType.DMA,
        pltpu.SemaphoreType.DMA,
    ],
    compiler_params=_sc_params,
)
def _pass_b(hn, src2, dst2, ec_in, norm0, norm1, agg0, agg1,
            ixd, ixs, bb0, bb1, nm0, nm1, nt0, nt1, ec0, ec1, mb0, mb1,
            rcp_sh, agg_sh, sa0, sa1, sn0, sn1, se0, se1, sc0, sc1):
    cid = lax.axis_index("c")
    sid = lax.axis_index("s")
    nch = jnp.where(cid == 0, NCH0B, NCH1B)
    cbase = jnp.where(cid == 0, sid * NCH0B, NS * NCH0B + sid * NCH1B)
    iota = lax.iota(jnp.int32, L)
    zero16 = jnp.zeros((L,), jnp.float32)
    rot = [(iota + d) % L for d in range(L)]
    rs = sid * RPS

    pltpu.sync_copy(dst2.at[pl.ds(cbase, NCH0B)], ixd)
    pltpu.sync_copy(src2.at[pl.ds(cbase, NCH0B)], ixs)

    for i in range(RPS // C):
        pltpu.sync_copy(norm0.at[pl.ds(rs + i * C, C)], mb0)
        pltpu.sync_copy(norm1.at[pl.ds(rs + i * C, C)], nm0)

        def rcprow(r, carry):
            mb0[r] = 1.0 / (mb0[r] + nm0[r] + 1e-12)
            return carry
        lax.fori_loop(0, C, rcprow, 0)
        pltpu.sync_copy(mb0, rcp_sh.at[pl.ds(rs + i * C, C)])

    def zrow(i, carry):
        mb0[i] = zero16
        return carry
    lax.fori_loop(0, C, zrow, 0)
    for i in range(RPS // C):
        pltpu.sync_copy(mb0, agg_sh.at[pl.ds(rs + i * C, C)])
    plsc.subcore_barrier()

    def issue(j, bb, nm, ec, sa, sn, se):
        pltpu.async_copy(hn.at[ixs.at[j]], bb, sa)
        pltpu.async_copy(rcp_sh.at[ixd.at[j]], nm, sn)
        pltpu.async_copy(ec_in.at[cbase + j], ec, se)

    def wait(j, bb, nm, ec, sa, sn, se):
        pltpu.make_async_copy(hn.at[ixs.at[j]], bb, sa).wait()
        pltpu.make_async_copy(rcp_sh.at[ixd.at[j]], nm, sn).wait()
        pltpu.make_async_copy(ec_in.at[cbase + j], ec, se).wait()

    def compute(bb, nm, nt, ec, mb):
        def group(g, gc):
            rows = g * L + iota
            for d in range(L):
                v = plsc.load_gather(nm, [rows, rot[d]])
                plsc.store_scatter(nt, [rot[d], rows], v)
            als = []
            for k in range(NV):
                s = pl.ds(g * L, L)
                als.append(ec[k, s] * nt[k, s] * (1.0 / NV))
            accs = [zero16] * DV
            for k in range(NV):
                for d in range(DV):
                    cv = k * DV + rot[d]
                    accs[d] = accs[d] + als[k] * plsc.load_gather(bb, [rows, cv])
            for d in range(DV):
                plsc.store_scatter(mb, [rows, rot[d]], accs[d])
            return gc
        lax.fori_loop(0, C // L, group, 0)

    issue(0, bb0, nm0, ec0, sa0, sn0, se0)
    issue(1, bb1, nm1, ec1, sa1, sn1, se1)

    def pair(jj, carry):
        c0 = 2 * jj
        c1 = c0 + 1
        wait(c0, bb0, nm0, ec0, sa0, sn0, se0)

        @pl.when(jj > 0)
        def _():
            pltpu.make_async_copy(mb0, agg_sh.at[ixd.at[c0 - 2]], sc0).wait()
        compute(bb0, nm0, nt0, ec0, mb0)
        pltpu.async_copy(mb0, agg_sh.at[ixd.at[c0]], sc0, add=True)

        @pl.when(c0 + 2 < nch)
        def _():
            issue(c0 + 2, bb0, nm0, ec0, sa0, sn0, se0)

        wait(c1, bb1, nm1, ec1, sa1, sn1, se1)

        @pl.when(jj > 0)
        def _():
            pltpu.make_async_copy(mb1, agg_sh.at[ixd.at[c1 - 2]], sc1).wait()
        compute(bb1, nm1, nt1, ec1, mb1)
        pltpu.async_copy(mb1, agg_sh.at[ixd.at[c1]], sc1, add=True)

        @pl.when(c1 + 2 < nch)
        def _():
            issue(c1 + 2, bb1, nm1, ec1, sa1, sn1, se1)
        return carry
    lax.fori_loop(0, nch // 2, pair, 0)
    pltpu.make_async_copy(mb0, agg_sh.at[ixd.at[nch - 2]], sc0).wait()
    pltpu.make_async_copy(mb1, agg_sh.at[ixd.at[nch - 1]], sc1).wait()

    plsc.subcore_barrier()

    @pl.when(cid == 0)
    def _():
        pltpu.sync_copy(agg_sh.at[pl.ds(rs, RPS)], agg0.at[pl.ds(rs, RPS)])

    @pl.when(cid == 1)
    def _():
        pltpu.sync_copy(agg_sh.at[pl.ds(rs, RPS)], agg1.at[pl.ds(rs, RPS)])


def _out_body(a0_ref, a1_ref, w_ref, b_ref, o_ref):
    agg = a0_ref[...] + a1_ref[...]
    o_ref[...] = jnp.maximum(
        jnp.dot(agg, w_ref[...], preferred_element_type=jnp.float32)
        + b_ref[...], 0.0)


def _final(a0, a1, W_out, b_out):
    BR = 512
    full = lambda i: (0, 0)
    row = lambda i: (i, 0)
    return pl.pallas_call(
        _out_body,
        grid=(NP // BR,),
        in_specs=[
            pl.BlockSpec((BR, L), row),
            pl.BlockSpec((BR, L), row),
            pl.BlockSpec((DV, DOUT), full),
            pl.BlockSpec((1, DOUT), full),
        ],
        out_specs=pl.BlockSpec((BR, DOUT), row),
        out_shape=jax.ShapeDtypeStruct((NP, DOUT), jnp.float32),
    )(a0, a1, W_out, b_out.reshape(1, -1))


def kernel(x, edge_index, W_target, b_target, W_source, b_source,
           W_hidden, b_hidden, W_out, b_out):
    xp = jnp.pad(x, ((0, NP - N), (0, 0)))
    ht, hs, hn = _project(
        xp, W_target, W_source, W_hidden,
        b_target.reshape(1, -1), b_source.reshape(1, -1),
        b_hidden.reshape(1, -1))

    ne = edge_index.shape[1]
    loops = jnp.arange(N, dtype=jnp.int32)
    padi = jnp.full((EPAD - ne - N + EXTRA,), NP - 1, jnp.int32)
    src2 = jnp.concatenate(
        [edge_index[0].astype(jnp.int32), loops, padi]).reshape(-1, C)
    dst2 = jnp.concatenate(
        [edge_index[1].astype(jnp.int32), loops, padi]).reshape(-1, C)

    ec, n0, n1 = _pass_a(ht, hs, src2, dst2)
    a0, a1 = _pass_b(hn, src2, dst2, ec, n0, n1)
    out = _final(a0, a1, W_out, b_out)
    return out[:N]


# speedup vs baseline: 1.2577x; 1.0209x over previous
import functools

import jax
import jax.numpy as jnp
from jax import lax
from jax.experimental import pallas as pl
from jax.experimental.pallas import tpu as pltpu
from jax.experimental.pallas import tpu_sc as plsc

N = 10000
D = 128
NV = 8
DV = 16
DOUT = 128
L = 16
NC = 2
NS = 16
NW = NC * NS
NP = 10240
C = 128
NCH0A = 112
NCH0B = 112
NCHT = 164
NCH1A = NCHT - NCH0A
NCH1B = NCHT - NCH0B
EPAD = NS * NCHT * C
EXTRA = max(NCH0A, NCH0B) * C
RPS = NP // NS

_mesh = plsc.VectorSubcoreMesh(core_axis_name="c", subcore_axis_name="s")
_sc_params = pltpu.CompilerParams(
    needs_layout_passes=False, use_tc_tiling_on_sc=False)


def _proj_body(x_ref, wt_ref, ws_ref, wh_ref, bt_ref, bs_ref, bh_ref,
               ht_ref, hs_ref, hn_ref):
    xv = x_ref[...]
    ht_ref[...] = jnp.maximum(
        jnp.dot(xv, wt_ref[...], preferred_element_type=jnp.float32)
        + bt_ref[...], 0.0)
    hs_ref[...] = jnp.maximum(
        jnp.dot(xv, ws_ref[...], preferred_element_type=jnp.float32)
        + bs_ref[...], 0.0)
    hn_ref[...] = jnp.maximum(
        jnp.dot(xv, wh_ref[...], preferred_element_type=jnp.float32)
        + bh_ref[...], 0.0)


def _project(xp, Wt, Ws, Wh, bt, bs, bh):
    BR = 512
    full = lambda i: (0, 0)
    row = lambda i: (i, 0)
    return pl.pallas_call(
        _proj_body,
        grid=(NP // BR,),
        in_specs=[
            pl.BlockSpec((BR, D), row),
            pl.BlockSpec((D, NV * DV), full),
            pl.BlockSpec((D, NV * DV), full),
            pl.BlockSpec((D, NV * DV), full),
            pl.BlockSpec((1, NV * DV), full),
            pl.BlockSpec((1, NV * DV), full),
            pl.BlockSpec((1, NV * DV), full),
        ],
        out_specs=[pl.BlockSpec((BR, NV * DV), row)] * 3,
        out_shape=[jax.ShapeDtypeStruct((NP, NV * DV), jnp.float32)] * 3,
    )(xp, Wt, Ws, Wh, bt, bs, bh)


@functools.partial(
    pl.kernel,
    out_type=[
        jax.ShapeDtypeStruct((NS * NCHT, L, C), jnp.float32),
        jax.ShapeDtypeStruct((NP, L), jnp.float32),
        jax.ShapeDtypeStruct((NP, L), jnp.float32),
    ],
    mesh=_mesh,
    scratch_types=[
        pltpu.VMEM((NCH0A, C), jnp.int32),
        pltpu.VMEM((NCH0A, C), jnp.int32),
        pltpu.VMEM((C, D), jnp.float32),
        pltpu.VMEM((C, D), jnp.float32),
        pltpu.VMEM((C, D), jnp.float32),
        pltpu.VMEM((C, D), jnp.float32),
        pltpu.VMEM((L, C), jnp.float32),
        pltpu.VMEM((L, C), jnp.float32),
        pltpu.VMEM((C, L), jnp.float32),
        pltpu.VMEM((C, L), jnp.float32),
        pltpu.VMEM_SHARED((NP, L), jnp.float32),
        pltpu.SemaphoreType.DMA,
        pltpu.SemaphoreType.DMA,
        pltpu.SemaphoreType.DMA,
        pltpu.SemaphoreType.DMA,
        pltpu.SemaphoreType.DMA,
        pltpu.SemaphoreType.DMA,
        pltpu.SemaphoreType.DMA,
        pltpu.SemaphoreType.DMA,
    ],
    compiler_params=_sc_params,
)
def _pass_a(ht, hs, src2, dst2, ec_out, norm0, norm1,
            ixd, ixs, ab0, ab1, bb0, bb1, et0, et1, rb0, rb1, norm_sh,
            sa0, sa1, sb0, sb1, se0, se1, sc0, sc1):
    cid = lax.axis_index("c")
    sid = lax.axis_index("s")
    nch = jnp.where(cid == 0, NCH0A, NCH1A)
    cbase = jnp.where(cid == 0, sid * NCH0A, NS * NCH0A + sid * NCH1A)
    iota = lax.iota(jnp.int32, L)
    zero16 = jnp.zeros((L,), jnp.float32)
    rot = [(iota + d) % L for d in range(L)]

    pltpu.sync_copy(dst2.at[pl.ds(cbase, NCH0A)], ixd)
    pltpu.sync_copy(src2.at[pl.ds(cbase, NCH0A)], ixs)

    def zrow(i, carry):
        rb0[i] = zero16
        return carry
    lax.fori_loop(0, C, zrow, 0)
    for i in range(RPS // C):
        pltpu.sync_copy(rb0, norm_sh.at[pl.ds(sid * RPS + i * C, C)])
    for et in (et0, et1):
        def zpad(g, carry):
            for k in range(NV, L):
                et[k, pl.ds(g * L, L)] = zero16
            return carry
        lax.fori_loop(0, C // L, zpad, 0)
    plsc.subcore_barrier()

    def issue(j, ab, bb, sa, sb):
        pltpu.async_copy(ht.at[ixd.at[j]], ab, sa)
        pltpu.async_copy(hs.at[ixs.at[j]], bb, sb)

    def wait(j, ab, bb, sa, sb):
        pltpu.make_async_copy(ht.at[ixd.at[j]], ab, sa).wait()
        pltpu.make_async_copy(hs.at[ixs.at[j]], bb, sb).wait()

    def compute(ab, bb, et, rb):
        def group(g, gc):
            rows = g * L + iota
            cols = g * L + iota
            for k in range(NV):
                acc = zero16
                for d in range(DV):
                    cv = k * DV + rot[d]
                    acc = acc + (plsc.load_gather(ab, [rows, cv])
                                 * plsc.load_gather(bb, [rows, cv]))
                et[k, pl.ds(g * L, L)] = jnp.exp(acc)
            for d in range(L):
                v = plsc.load_gather(et, [rot[d], cols])
                plsc.store_scatter(rb, [cols, rot[d]], v)
            return gc
        lax.fori_loop(0, C // L, group, 0)

    def writeout(j, et, rb, se, sc):
        pltpu.async_copy(et, ec_out.at[cbase + j], se)
        pltpu.async_copy(rb, norm_sh.at[ixd.at[j]], sc, add=True)

    def drain(j, et, rb, se, sc):
        pltpu.make_async_copy(et, ec_out.at[cbase + j], se).wait()
        pltpu.make_async_copy(rb, norm_sh.at[ixd.at[j]], sc).wait()

    issue(0, ab0, bb0, sa0, sb0)
    issue(1, ab1, bb1, sa1, sb1)

    def pair(jj, carry):
        c0 = 2 * jj
        c1 = c0 + 1
        wait(c0, ab0, bb0, sa0, sb0)

        @pl.when(jj > 0)
        def _():
            drain(c0 - 2, et0, rb0, se0, sc0)
        compute(ab0, bb0, et0, rb0)
        writeout(c0, et0, rb0, se0, sc0)

        @pl.when(c0 + 2 < nch)
        def _():
            issue(c0 + 2, ab0, bb0, sa0, sb0)

        wait(c1, ab1, bb1, sa1, sb1)

        @pl.when(jj > 0)
        def _():
            drain(c1 - 2, et1, rb1, se1, sc1)
        compute(ab1, bb1, et1, rb1)
        writeout(c1, et1, rb1, se1, sc1)

        @pl.when(c1 + 2 < nch)
        def _():
            issue(c1 + 2, ab1, bb1, sa1, sb1)
        return carry
    lax.fori_loop(0, nch // 2, pair, 0)
    drain(nch - 2, et0, rb0, se0, sc0)
    drain(nch - 1, et1, rb1, se1, sc1)

    plsc.subcore_barrier()
    rs = sid * RPS

    @pl.when(cid == 0)
    def _():
        pltpu.sync_copy(norm_sh.at[pl.ds(rs, RPS)], norm0.at[pl.ds(rs, RPS)])

    @pl.when(cid == 1)
    def _():
        pltpu.sync_copy(norm_sh.at[pl.ds(rs, RPS)], norm1.at[pl.ds(rs, RPS)])


@functools.partial(
    pl.kernel,
    out_type=[
        jax.ShapeDtypeStruct((NP, L), jnp.float32),
        jax.ShapeDtypeStruct((NP, L), jnp.float32),
    ],
    mesh=_mesh,
    scratch_types=[
        pltpu.VMEM((NCH0B, C), jnp.int32),
        pltpu.VMEM((NCH0B, C), jnp.int32),
        pltpu.VMEM((C, D), jnp.float32),
        pltpu.VMEM((C, D), jnp.float32),
        pltpu.VMEM((C, L), jnp.float32),
        pltpu.VMEM((C, L), jnp.float32),
        pltpu.VMEM((L, C), jnp.float32),
        pltpu.VMEM((L, C), jnp.float32),
        pltpu.VMEM((L, C), jnp.float32),
        pltpu.VMEM((L, C), jnp.float32),
        pltpu.VMEM((C, L), jnp.float32),
        pltpu.VMEM((C, L), jnp.float32),
        pltpu.VMEM_SHARED((NP, L), jnp.float32),
        pltpu.VMEM_SHARED((NP, L), jnp.float32),
        pltpu.SemaphoreType.DMA,
        pltpu.SemaphoreType.DMA,
        pltpu.SemaphoreType.DMA,
        pltpu.SemaphoreType.DMA,
        pltpu.SemaphoreType.DMA,
        pltpu.SemaphoreType.DMA,
        pltpu.SemaphoreType.DMA,
        pltpu.SemaphoreType.DMA,
    ],
    compiler_params=_sc_params,
)
def _pass_b(hn, src2, dst2, ec_in, norm0, norm1, agg0, agg1,
            ixd, ixs, bb0, bb1, nm0, nm1, nt0, nt1, ec0, ec1, mb0, mb1,
            rcp_sh, agg_sh, sa0, sa1, sn0, sn1, se0, se1, sc0, sc1):
    cid = lax.axis_index("c")
    sid = lax.axis_index("s")
    nch = jnp.where(cid == 0, NCH0B, NCH1B)
    cbase = jnp.where(cid == 0, sid * NCH0B, NS * NCH0B + sid * NCH1B)
    iota = lax.iota(jnp.int32, L)
    zero16 = jnp.zeros((L,), jnp.float32)
    rot = [(iota + d) % L for d in range(L)]
    rs = sid * RPS

    pltpu.sync_copy(dst2.at[pl.ds(cbase, NCH0B)], ixd)
    pltpu.sync_copy(src2.at[pl.ds(cbase, NCH0B)], ixs)

    for i in range(RPS // C):
        pltpu.sync_copy(norm0.at[pl.ds(rs + i * C, C)], mb0)
        pltpu.sync_copy(norm1.at[pl.ds(rs + i * C, C)], nm0)

        def rcprow(r, carry):
            mb0[r] = 1.0 / (mb0[r] + nm0[r] + 1e-12)
            return carry
        lax.fori_loop(0, C, rcprow, 0)
        pltpu.sync_copy(mb0, rcp_sh.at[pl.ds(rs + i * C, C)])

    def zrow(i, carry):
        mb0[i] = zero16
        return carry
    lax.fori_loop(0, C, zrow, 0)
    for i in range(RPS // C):
        pltpu.sync_copy(mb0, agg_sh.at[pl.ds(rs + i * C, C)])
    plsc.subcore_barrier()

    def issue(j, bb, nm, ec, sa, sn, se):
        pltpu.async_copy(hn.at[ixs.at[j]], bb, sa)
        pltpu.async_copy(rcp_sh.at[ixd.at[j]], nm, sn)
        pltpu.async_copy(ec_in.at[cbase + j], ec, se)

    def wait(j, bb, nm, ec, sa, sn, se):
        pltpu.make_async_copy(hn.at[ixs.at[j]], bb, sa).wait()
        pltpu.make_async_copy(rcp_sh.at[ixd.at[j]], nm, sn).wait()
        pltpu.make_async_copy(ec_in.at[cbase + j], ec, se).wait()

    def compute(bb, nm, nt, ec, mb):
        def group(g, gc):
            rows = g * L + iota
            for d in range(L):
                v = plsc.load_gather(nm, [rows, rot[d]])
                plsc.store_scatter(nt, [rot[d], rows], v)
            als = []
            for k in range(NV):
                s = pl.ds(g * L, L)
                als.append(ec[k, s] * nt[k, s] * (1.0 / NV))
            accs = [zero16] * DV
            for k in range(NV):
                for d in range(DV):
                    cv = k * DV + rot[d]
                    accs[d] = accs[d] + als[k] * plsc.load_gather(bb, [rows, cv])
            for d in range(DV):
                plsc.store_scatter(mb, [rows, rot[d]], accs[d])
            return gc
        lax.fori_loop(0, C // L, group, 0)

    issue(0, bb0, nm0, ec0, sa0, sn0, se0)
    issue(1, bb1, nm1, ec1, sa1, sn1, se1)

    def pair(jj, carry):
        c0 = 2 * jj
        c1 = c0 + 1
        wait(c0, bb0, nm0, ec0, sa0, sn0, se0)

        @pl.when(jj > 0)
        def _():
            pltpu.make_async_copy(mb0, agg_sh.at[ixd.at[c0 - 2]], sc0).wait()
        compute(bb0, nm0, nt0, ec0, mb0)
        pltpu.async_copy(mb0, agg_sh.at[ixd.at[c0]], sc0, add=True)

        @pl.when(c0 + 2 < nch)
        def _():
            issue(c0 + 2, bb0, nm0, ec0, sa0, sn0, se0)

        wait(c1, bb1, nm1, ec1, sa1, sn1, se1)

        @pl.when(jj > 0)
        def _():
            pltpu.make_async_copy(mb1, agg_sh.at[ixd.at[c1 - 2]], sc1).wait()
        compute(bb1, nm1, nt1, ec1, mb1)
        pltpu.async_copy(mb1, agg_sh.at[ixd.at[c1]], sc1, add=True)

        @pl.when(c1 + 2 < nch)
        def _():
            issue(c1 + 2, bb1, nm1, ec1, sa1, sn1, se1)
        return carry
    lax.fori_loop(0, nch // 2, pair, 0)
    pltpu.make_async_copy(mb0, agg_sh.at[ixd.at[nch - 2]], sc0).wait()
    pltpu.make_async_copy(mb1, agg_sh.at[ixd.at[nch - 1]], sc1).wait()

    plsc.subcore_barrier()

    @pl.when(cid == 0)
    def _():
        pltpu.sync_copy(agg_sh.at[pl.ds(rs, RPS)], agg0.at[pl.ds(rs, RPS)])

    @pl.when(cid == 1)
    def _():
        pltpu.sync_copy(agg_sh.at[pl.ds(rs, RPS)], agg1.at[pl.ds(rs, RPS)])


def _out_body(a0_ref, a1_ref, w_ref, b_ref, o_ref):
    agg = a0_ref[...] + a1_ref[...]
    o_ref[...] = jnp.maximum(
        jnp.dot(agg, w_ref[...], preferred_element_type=jnp.float32)
        + b_ref[...], 0.0)


def _final(a0, a1, W_out, b_out):
    BR = 512
    full = lambda i: (0, 0)
    row = lambda i: (i, 0)
    return pl.pallas_call(
        _out_body,
        grid=(NP // BR,),
        in_specs=[
            pl.BlockSpec((BR, L), row),
            pl.BlockSpec((BR, L), row),
            pl.BlockSpec((DV, DOUT), full),
            pl.BlockSpec((1, DOUT), full),
        ],
        out_specs=pl.BlockSpec((BR, DOUT), row),
        out_shape=jax.ShapeDtypeStruct((NP, DOUT), jnp.float32),
    )(a0, a1, W_out, b_out.reshape(1, -1))


def kernel(x, edge_index, W_target, b_target, W_source, b_source,
           W_hidden, b_hidden, W_out, b_out):
    xp = jnp.pad(x, ((0, NP - N), (0, 0)))
    ht, hs, hn = _project(
        xp, W_target, W_source, W_hidden,
        b_target.reshape(1, -1), b_source.reshape(1, -1),
        b_hidden.reshape(1, -1))

    ne = edge_index.shape[1]
    loops = jnp.arange(N, dtype=jnp.int32)
    padi = jnp.full((EPAD - ne - N + EXTRA,), NP - 1, jnp.int32)
    src2 = jnp.concatenate(
        [edge_index[0].astype(jnp.int32), loops, padi]).reshape(-1, C)
    dst2 = jnp.concatenate(
        [edge_index[1].astype(jnp.int32), loops, padi]).reshape(-1, C)

    ec, n0, n1 = _pass_a(ht, hs, src2, dst2)
    a0, a1 = _pass_b(hn, src2, dst2, ec, n0, n1)
    out = _final(a0, a1, W_out, b_out)
    return out[:N]
